# baseline (speedup 1.0000x reference)
import dataclasses
import functools
import math

import jax
import jax.numpy as jnp
from jax.experimental import pallas as pl
from jax.experimental.pallas import tpu as pltpu
from jax.experimental.pallas import tpu_sc as plsc

_CUTOFF = 5.0
_NUM_BASIS = 16
_LANES = 128
_BLOCK_ROWS = 2000


def _edge_body(a_ref, o_ref):
    a = a_ref[...]
    sq = a * a
    ii = jax.lax.broadcasted_iota(jnp.int32, (24, _LANES), 0)
    ll = jax.lax.broadcasted_iota(jnp.int32, (24, _LANES), 1)
    B = (ii // 3 == ll // _NUM_BASIS).astype(jnp.float32)
    r2 = jax.lax.dot_general(
        sq, B, (((1,), (0,)), ((), ())),
        precision=jax.lax.Precision.HIGHEST,
        preferred_element_type=jnp.float32,
    )
    r = jnp.sqrt(r2)
    n = (jax.lax.broadcasted_iota(jnp.int32, r.shape, 1) % _NUM_BASIS + 1
         ).astype(jnp.float32)
    arg = n * (math.pi / _CUTOFF) * r
    o_ref[...] = (math.sqrt(2.0 / _CUTOFF) / r) * jnp.sin(arg)


def _edge_call(a24):
    rows = a24.shape[0]
    grid = rows // _BLOCK_ROWS
    return pl.pallas_call(
        _edge_body,
        grid=(grid,),
        in_specs=[pl.BlockSpec((_BLOCK_ROWS, 24), lambda i: (i, 0))],
        out_specs=pl.BlockSpec((_BLOCK_ROWS, _LANES), lambda i: (i, 0)),
        out_shape=jax.ShapeDtypeStruct((rows, _LANES), jnp.float32),
    )(a24)


def _node_gather(x_idx, W_x, W_z):
    n_real = x_idx.shape[0]
    n_workers = 32
    b_per_w = 3200
    B = n_workers * b_per_w
    idx = jnp.zeros((B,), x_idx.dtype).at[:n_real].set(x_idx)
    table = jnp.concatenate([W_x, W_z], axis=1)
    mesh = plsc.VectorSubcoreMesh(core_axis_name="c", subcore_axis_name="s")

    cp = pltpu.CompilerParams()
    if "needs_layout_passes" in pltpu.CompilerParams.__dataclass_fields__:
        cp = dataclasses.replace(cp, needs_layout_passes=False)

    @functools.partial(
        pl.kernel, mesh=mesh, compiler_params=cp,
        out_type=jax.ShapeDtypeStruct((B * 16,), jnp.float32),
        scratch_types=[pltpu.VMEM((b_per_w,), jnp.int32),
                       pltpu.VMEM((b_per_w * 16,), jnp.float32),
                       pltpu.VMEM((100, 16), jnp.float32)])
    def knl(table_hbm, idx_hbm, out_hbm, idx_v, rows_v, tab_v):
        wid = jax.lax.axis_index("s") * 2 + jax.lax.axis_index("c")
        base = wid * b_per_w
        pltpu.sync_copy(table_hbm, tab_v)
        pltpu.sync_copy(idx_hbm.at[pl.ds(base, b_per_w)], idx_v)
        cols = jax.lax.iota(jnp.int32, 16)

        @pl.loop(0, b_per_w)
        def _(k):
            iv = plsc.load_gather(idx_v, [jnp.full((16,), k, jnp.int32)])
            vals = plsc.load_gather(tab_v, [iv, cols])
            rows_v[pl.ds(k * 16, 16)] = vals

        pltpu.sync_copy(rows_v, out_hbm.at[pl.ds(base * 16, b_per_w * 16)])

    out = knl(table, idx).reshape(B, 16)
    return out[:n_real, :8], out[:n_real, 8:]


def kernel(x, edge_attr, W_x, W_z):
    E = edge_attr.shape[0]
    h_edge = _edge_call(edge_attr.reshape(E // 8, 24)).reshape(E, _NUM_BASIS)
    h_node_x, h_node_z = _node_gather(x, W_x, W_z)
    return (h_node_x, h_node_z, h_edge)


# speedup vs baseline: 3.1530x; 3.1530x over previous
import dataclasses
import functools
import math

import jax
import jax.numpy as jnp
from jax.experimental import pallas as pl
from jax.experimental.pallas import tpu as pltpu
from jax.experimental.pallas import tpu_sc as plsc

_CUTOFF = 5.0
_NUM_BASIS = 16
_LANES = 128
_BLOCK_ROWS = 8000


_S1 = 3.1415917330
_S3 = -5.1676850392
_S5 = 2.5499267721
_S7 = -5.9839777752e-1
_S9 = 8.0605215494e-2
_S11 = -6.0412088560e-3


def _edge_body(a_ref, o_ref):
    a = a_ref[...]
    sq = a * a
    ones = jnp.ones((3, _NUM_BASIS), jnp.float32)
    r2 = jax.lax.dot_general(
        sq, ones, (((1,), (0,)), ((), ())),
        precision=jax.lax.Precision.HIGHEST,
        preferred_element_type=jnp.float32,
    )
    inv_r = jax.lax.rsqrt(r2)
    n_over_c = ((jax.lax.broadcasted_iota(jnp.int32, r2.shape, 1) + 1)
                .astype(jnp.float32)) * (1.0 / _CUTOFF)
    t = (r2 * inv_r) * n_over_c
    m = t - 2.0 * jnp.round(t * 0.5)
    m2 = m * m
    p = _S11
    p = p * m2 + _S9
    p = p * m2 + _S7
    p = p * m2 + _S5
    p = p * m2 + _S3
    p = p * m2 + _S1
    o_ref[...] = (p * m) * (math.sqrt(2.0 / _CUTOFF) * inv_r)


def _edge_call(edge_attr):
    rows = edge_attr.shape[0]
    grid = rows // _BLOCK_ROWS
    return pl.pallas_call(
        _edge_body,
        grid=(grid,),
        in_specs=[pl.BlockSpec((_BLOCK_ROWS, 3), lambda i: (i, 0))],
        out_specs=pl.BlockSpec((_BLOCK_ROWS, _NUM_BASIS), lambda i: (i, 0)),
        out_shape=jax.ShapeDtypeStruct((rows, _NUM_BASIS), jnp.float32),
    )(edge_attr)


def _node_gather(x_idx, W_x, W_z):
    n_real = x_idx.shape[0]
    n_workers = 32
    b_per_w = 3200
    B = n_workers * b_per_w
    idx = jnp.zeros((B,), x_idx.dtype).at[:n_real].set(x_idx)
    table = jnp.concatenate([W_x, W_z], axis=1)
    mesh = plsc.VectorSubcoreMesh(core_axis_name="c", subcore_axis_name="s")

    cp = pltpu.CompilerParams()
    if "needs_layout_passes" in pltpu.CompilerParams.__dataclass_fields__:
        cp = dataclasses.replace(cp, needs_layout_passes=False)

    @functools.partial(
        pl.kernel, mesh=mesh, compiler_params=cp,
        out_type=jax.ShapeDtypeStruct((B * 16,), jnp.float32),
        scratch_types=[pltpu.VMEM((b_per_w,), jnp.int32),
                       pltpu.VMEM((b_per_w * 16,), jnp.float32),
                       pltpu.VMEM((100, 16), jnp.float32)])
    def knl(table_hbm, idx_hbm, out_hbm, idx_v, rows_v, tab_v):
        wid = jax.lax.axis_index("s") * 2 + jax.lax.axis_index("c")
        base = wid * b_per_w
        pltpu.sync_copy(table_hbm, tab_v)
        pltpu.sync_copy(idx_hbm.at[pl.ds(base, b_per_w)], idx_v)
        cols = jax.lax.iota(jnp.int32, 16)

        @pl.loop(0, b_per_w)
        def _(k):
            iv = plsc.load_gather(idx_v, [jnp.full((16,), k, jnp.int32)])
            vals = plsc.load_gather(tab_v, [iv, cols])
            rows_v[pl.ds(k * 16, 16)] = vals

        pltpu.sync_copy(rows_v, out_hbm.at[pl.ds(base * 16, b_per_w * 16)])

    out = knl(table, idx).reshape(B, 16)
    return out[:n_real, :8], out[:n_real, 8:]


def kernel(x, edge_attr, W_x, W_z):
    h_edge = _edge_call(edge_attr)
    h_node_x, h_node_z = _node_gather(x, W_x, W_z)
    return (h_node_x, h_node_z, h_edge)
